# Initial kernel scaffold; baseline (speedup 1.0000x reference)
import jax
import jax.numpy as jnp
from jax.experimental import pallas as pl


def kernel(boxes, scores):
    raise NotImplementedError("write your pallas kernel here")


# speedup vs baseline: 8.8806x; 8.8806x over previous
import jax
import jax.numpy as jnp
from jax.experimental import pallas as pl
from jax.experimental.pallas import tpu as pltpu

_N = 20000
_NP = 20480
_ROWS = 10
_LANES = 2048
_K = 2000
_KP = 2048
_OUT = 1000
_OUTP = 1024
_TH = 0.7
_NEG = -1e30
_ST = 256
_NS = _KP // _ST
_HI = jax.lax.Precision.HIGHEST


def _dot(a, b, dims):
    return jax.lax.dot_general(a, b, (dims, ((), ())),
                               preferred_element_type=jnp.float32,
                               precision=_HI)


def _nms_kernel(scores_ref, payload_ref, out_ref, sup_ref, keep_ref,
                comp_ref, srt_ref):
    f32 = jnp.float32
    s = scores_ref[...]
    b = jax.lax.bitcast_convert_type(s, jnp.int32)
    key = b ^ ((b >> 31) & jnp.int32(0x7FFFFFFF))

    def bs_body(_, lohi):
        lo, hi = lohi
        x = lo ^ hi
        mid = (lo & hi) + (x >> 1) + (x & 1)
        cnt = jnp.sum((key >= mid).astype(jnp.int32))
        ok = cnt >= _K
        return jnp.where(ok, mid, lo), jnp.where(ok, hi, mid - 1)

    v, _ = jax.lax.fori_loop(
        0, 32, bs_body, (jnp.int32(-2147483647 - 1), jnp.int32(2147483647)))

    c_gt = jnp.sum((key > v).astype(jnp.int32))
    t = _K - c_gt
    idx = (jax.lax.broadcasted_iota(jnp.int32, (_ROWS, _LANES), 0) * _LANES
           + jax.lax.broadcasted_iota(jnp.int32, (_ROWS, _LANES), 1))
    eqv = key == v

    def bs2_body(_, lohi):
        lo, hi = lohi
        mid = (lo & hi) + ((lo ^ hi) >> 1)
        cnt = jnp.sum((eqv & (idx <= mid)).astype(jnp.int32))
        ok = cnt >= t
        return jnp.where(ok, lo, mid + 1), jnp.where(ok, mid, hi)

    icut, _ = jax.lax.fori_loop(
        0, 15, bs2_body, (jnp.int32(0), jnp.int32(_NP - 1)))

    sel = ((key > v) | (eqv & (idx <= icut))).astype(f32)

    irow = jax.lax.broadcasted_iota(jnp.int32, (1, _KP), 1)

    def strip_iota(rs):
        return (jax.lax.broadcasted_iota(jnp.int32, (_ST, 1), 0) + rs * _ST)

    within = jnp.zeros((_ROWS, _LANES), f32)
    for rs in range(_NS):
        ustrip = (strip_iota(rs) < irow).astype(f32)
        within = within + _dot(sel[:, rs * _ST:(rs + 1) * _ST], ustrip,
                               ((1,), (0,)))
    rowsum = jnp.sum(sel, axis=1, keepdims=True)
    r10 = jax.lax.broadcasted_iota(jnp.int32, (_ROWS, _ROWS), 0)
    c10 = jax.lax.broadcasted_iota(jnp.int32, (_ROWS, _ROWS), 1)
    low10 = (c10 < r10).astype(f32)
    off = _dot(low10, rowsum, ((1,), (0,)))
    pos = within + off

    payload = payload_ref[...]
    for rs in range(_NS):
        rcol_s = strip_iota(rs).astype(f32)
        acc = jnp.zeros((_ST, 8), f32)
        for c in range(_ROWS):
            oh = ((pos[c:c + 1, :] == rcol_s)
                  & (sel[c:c + 1, :] > 0.5)).astype(f32)
            acc = acc + _dot(oh, payload[c * _LANES:(c + 1) * _LANES, :],
                             ((1,), (0,)))
        comp_ref[rs * _ST:(rs + 1) * _ST, :] = acc
    comp = comp_ref[...]

    comp_t = jnp.zeros((8, _KP), f32)
    for rs in range(_NS):
        estrip = (strip_iota(rs) == irow).astype(f32)
        comp_t = comp_t + _dot(comp[rs * _ST:(rs + 1) * _ST, :], estrip,
                               ((0,), (0,)))

    rrow = irow.astype(f32)
    sc_r = jnp.where(rrow < float(_K), comp_t[4:5, :], _NEG)
    ix_r = jnp.where(rrow < float(_K), comp_t[5:6, :], 1e6 + rrow)
    rank = jnp.zeros((1, _KP), f32)
    for rs in range(_NS):
        rcol_s = strip_iota(rs).astype(f32)
        cs = comp[rs * _ST:(rs + 1) * _ST, :]
        sc_c = jnp.where(rcol_s < float(_K), cs[:, 4:5], _NEG)
        ix_c = jnp.where(rcol_s < float(_K), cs[:, 5:6], 1e6 + rcol_s)
        bef = ((sc_c > sc_r) | ((sc_c == sc_r) & (ix_c < ix_r))).astype(f32)
        rank = rank + jnp.sum(bef, axis=0, keepdims=True)

    for rs in range(_NS):
        rcol_s = strip_iota(rs).astype(f32)
        q = (rank == rcol_s).astype(f32)
        srt_ref[rs * _ST:(rs + 1) * _ST, :] = _dot(q, comp, ((1,), (0,)))
    srt = srt_ref[...]

    srt_t = jnp.zeros((8, _KP), f32)
    for rs in range(_NS):
        estrip = (strip_iota(rs) == irow).astype(f32)
        srt_t = srt_t + _dot(srt[rs * _ST:(rs + 1) * _ST, :], estrip,
                             ((0,), (0,)))

    x1r, y1r = srt_t[0:1, :], srt_t[1:2, :]
    x2r, y2r = srt_t[2:3, :], srt_t[3:4, :]
    area_r = (x2r - x1r) * (y2r - y1r)
    for rs in range(_NS):
        cs = srt[rs * _ST:(rs + 1) * _ST, :]
        x1c, y1c, x2c, y2c = cs[:, 0:1], cs[:, 1:2], cs[:, 2:3], cs[:, 3:4]
        area_c = (x2c - x1c) * (y2c - y1c)
        iw = jnp.maximum(jnp.minimum(x2c, x2r) - jnp.maximum(x1c, x1r), 0.0)
        ih = jnp.maximum(jnp.minimum(y2c, y2r) - jnp.maximum(y1c, y1r), 0.0)
        inter = iw * ih
        union = jnp.maximum(area_c + area_r - inter, 1e-9)
        sup_ref[rs * _ST:(rs + 1) * _ST, :] = (
            (inter / union > _TH) & (irow > strip_iota(rs))).astype(f32)

    keep_ref[...] = jnp.ones((1, _KP), f32)

    def nms_body(r, carry):
        srow = sup_ref[pl.ds(r, 1), :]
        keep_v = keep_ref[...]
        kr = jnp.sum(keep_v * (irow == r).astype(f32))
        keep_ref[...] = keep_v * (1.0 - srow * kr)
        return carry

    jax.lax.fori_loop(0, _K, nms_body, jnp.int32(0))
    keep = keep_ref[...]

    ds_r = jnp.where(rrow < float(_K), srt_t[4:5, :], _NEG)
    ks_r = jnp.where(keep > 0.5, ds_r, _NEG)
    frank = jnp.zeros((1, _KP), f32)
    for rs in range(_NS):
        rcol_s = strip_iota(rs).astype(f32)
        estrip = (strip_iota(rs) == irow).astype(f32)
        keep_c = _dot(estrip, keep, ((1,), (1,)))
        cs = srt[rs * _ST:(rs + 1) * _ST, :]
        ds_c = jnp.where(rcol_s < float(_K), cs[:, 4:5], _NEG)
        ks_c = jnp.where(keep_c > 0.5, ds_c, _NEG)
        bef = ((ks_c > ks_r) | ((ks_c == ks_r) & (rcol_s < rrow))).astype(f32)
        frank = frank + jnp.sum(bef, axis=0, keepdims=True)

    kept_cnt = jnp.sum(keep * (rrow < float(_K)).astype(f32))
    col = jax.lax.broadcasted_iota(jnp.int32, (_ST, 8), 1)
    for rs in range(_OUTP // _ST):
        rcol_s = strip_iota(rs).astype(f32)
        oh = (frank == rcol_s).astype(f32)
        o = _dot(oh, srt, ((1,), (0,)))
        fix = jnp.where(rcol_s < kept_cnt, o, -jnp.inf)
        out_ref[rs * _ST:(rs + 1) * _ST, :] = jnp.where(col == 4, fix, o)


def kernel(boxes, scores):
    sp = jnp.full((_NP,), _NEG, jnp.float32).at[:_N].set(scores)
    s2 = sp.reshape(_ROWS, _LANES)
    bp = jnp.zeros((_NP, 4), jnp.float32).at[:_N].set(boxes)
    idxf = jnp.arange(_NP, dtype=jnp.float32)[:, None]
    payload = jnp.concatenate(
        [bp, sp[:, None], idxf, jnp.zeros((_NP, 2), jnp.float32)], axis=1)
    out = pl.pallas_call(
        _nms_kernel,
        out_shape=jax.ShapeDtypeStruct((_OUTP, 8), jnp.float32),
        scratch_shapes=[pltpu.VMEM((_KP, _KP), jnp.float32),
                        pltpu.VMEM((1, _KP), jnp.float32),
                        pltpu.VMEM((_KP, 8), jnp.float32),
                        pltpu.VMEM((_KP, 8), jnp.float32)],
    )(s2, payload)
    return out[:_OUT, :5]


# speedup vs baseline: 9.6910x; 1.0913x over previous
import jax
import jax.numpy as jnp
from jax.experimental import pallas as pl
from jax.experimental.pallas import tpu as pltpu

_N = 20000
_NP = 20480
_ROWS = 10
_LANES = 2048
_K = 2000
_KP = 2048
_OUT = 1000
_OUTP = 1024
_TH = 0.7
_NEG = -1e30
_ST = 256
_NS = _KP // _ST
_HI = jax.lax.Precision.HIGHEST


def _dot(a, b, dims):
    return jax.lax.dot_general(a, b, (dims, ((), ())),
                               preferred_element_type=jnp.float32,
                               precision=_HI)


def _nms_kernel(scores_ref, payload_ref, out_ref, sup_ref, keep_ref,
                comp_ref, srt_ref, supd_ref):
    f32 = jnp.float32
    s = scores_ref[...]
    b = jax.lax.bitcast_convert_type(s, jnp.int32)
    key = b ^ ((b >> 31) & jnp.int32(0x7FFFFFFF))

    def bs_body(_, lohi):
        lo, hi = lohi
        x = lo ^ hi
        mid = (lo & hi) + (x >> 1) + (x & 1)
        cnt = jnp.sum((key >= mid).astype(jnp.int32))
        ok = cnt >= _K
        return jnp.where(ok, mid, lo), jnp.where(ok, hi, mid - 1)

    v, _ = jax.lax.fori_loop(
        0, 32, bs_body, (jnp.int32(-2147483647 - 1), jnp.int32(2147483647)))

    c_gt = jnp.sum((key > v).astype(jnp.int32))
    t = _K - c_gt
    idx = (jax.lax.broadcasted_iota(jnp.int32, (_ROWS, _LANES), 0) * _LANES
           + jax.lax.broadcasted_iota(jnp.int32, (_ROWS, _LANES), 1))
    eqv = key == v

    def bs2_body(_, lohi):
        lo, hi = lohi
        mid = (lo & hi) + ((lo ^ hi) >> 1)
        cnt = jnp.sum((eqv & (idx <= mid)).astype(jnp.int32))
        ok = cnt >= t
        return jnp.where(ok, lo, mid + 1), jnp.where(ok, mid, hi)

    icut, _ = jax.lax.fori_loop(
        0, 15, bs2_body, (jnp.int32(0), jnp.int32(_NP - 1)))

    sel = ((key > v) | (eqv & (idx <= icut))).astype(f32)

    irow = jax.lax.broadcasted_iota(jnp.int32, (1, _KP), 1)

    def strip_iota(rs):
        return (jax.lax.broadcasted_iota(jnp.int32, (_ST, 1), 0) + rs * _ST)

    within = jnp.zeros((_ROWS, _LANES), f32)
    for rs in range(_NS):
        ustrip = (strip_iota(rs) < irow).astype(f32)
        within = within + _dot(sel[:, rs * _ST:(rs + 1) * _ST], ustrip,
                               ((1,), (0,)))
    rowsum = jnp.sum(sel, axis=1, keepdims=True)
    r10 = jax.lax.broadcasted_iota(jnp.int32, (_ROWS, _ROWS), 0)
    c10 = jax.lax.broadcasted_iota(jnp.int32, (_ROWS, _ROWS), 1)
    low10 = (c10 < r10).astype(f32)
    off = _dot(low10, rowsum, ((1,), (0,)))
    pos = within + off

    payload = payload_ref[...]
    for rs in range(_NS):
        rcol_s = strip_iota(rs).astype(f32)
        acc = jnp.zeros((_ST, 8), f32)
        for c in range(_ROWS):
            oh = ((pos[c:c + 1, :] == rcol_s)
                  & (sel[c:c + 1, :] > 0.5)).astype(f32)
            acc = acc + _dot(oh, payload[c * _LANES:(c + 1) * _LANES, :],
                             ((1,), (0,)))
        comp_ref[rs * _ST:(rs + 1) * _ST, :] = acc
    comp = comp_ref[...]

    comp_t = jnp.zeros((8, _KP), f32)
    for rs in range(_NS):
        estrip = (strip_iota(rs) == irow).astype(f32)
        comp_t = comp_t + _dot(comp[rs * _ST:(rs + 1) * _ST, :], estrip,
                               ((0,), (0,)))

    rrow = irow.astype(f32)
    sc_r = jnp.where(rrow < float(_K), comp_t[4:5, :], _NEG)
    ix_r = jnp.where(rrow < float(_K), comp_t[5:6, :], 1e6 + rrow)
    rank = jnp.zeros((1, _KP), f32)
    for rs in range(_NS):
        rcol_s = strip_iota(rs).astype(f32)
        cs = comp[rs * _ST:(rs + 1) * _ST, :]
        sc_c = jnp.where(rcol_s < float(_K), cs[:, 4:5], _NEG)
        ix_c = jnp.where(rcol_s < float(_K), cs[:, 5:6], 1e6 + rcol_s)
        bef = ((sc_c > sc_r) | ((sc_c == sc_r) & (ix_c < ix_r))).astype(f32)
        rank = rank + jnp.sum(bef, axis=0, keepdims=True)

    for rs in range(_NS):
        rcol_s = strip_iota(rs).astype(f32)
        q = (rank == rcol_s).astype(f32)
        srt_ref[rs * _ST:(rs + 1) * _ST, :] = _dot(q, comp, ((1,), (0,)))
    srt = srt_ref[...]

    srt_t = jnp.zeros((8, _KP), f32)
    for rs in range(_NS):
        estrip = (strip_iota(rs) == irow).astype(f32)
        srt_t = srt_t + _dot(srt[rs * _ST:(rs + 1) * _ST, :], estrip,
                             ((0,), (0,)))

    x1r, y1r = srt_t[0:1, :], srt_t[1:2, :]
    x2r, y2r = srt_t[2:3, :], srt_t[3:4, :]
    area_r = (x2r - x1r) * (y2r - y1r)
    for rs in range(_NS):
        cs = srt[rs * _ST:(rs + 1) * _ST, :]
        x1c, y1c, x2c, y2c = cs[:, 0:1], cs[:, 1:2], cs[:, 2:3], cs[:, 3:4]
        area_c = (x2c - x1c) * (y2c - y1c)
        iw = jnp.maximum(jnp.minimum(x2c, x2r) - jnp.maximum(x1c, x1r), 0.0)
        ih = jnp.maximum(jnp.minimum(y2c, y2r) - jnp.maximum(y1c, y1r), 0.0)
        inter = iw * ih
        union = jnp.maximum(area_c + area_r - inter, 1e-9)
        supv = ((inter / union > _TH) & (irow > strip_iota(rs))).astype(f32)
        sup_ref[rs * _ST:(rs + 1) * _ST, :] = supv
        for h in range(_ST // 128):
            rb = rs * _ST + h * 128
            supd_ref[rb:rb + 128, :] = supv[h * 128:(h + 1) * 128,
                                            rb:rb + 128]

    _B = 128
    keep_ref[...] = jnp.ones((1, _KP), f32)
    iblk = jax.lax.broadcasted_iota(jnp.int32, (1, _B), 1)
    for blk in range(_KP // _B):
        cb = blk * _B

        def blk_body(r_local, kb):
            srow = supd_ref[pl.ds(cb + r_local, 1), :]
            kr = jnp.sum(kb * (iblk == r_local).astype(f32))
            return kb * (1.0 - srow * kr)

        kb = jax.lax.fori_loop(0, _B, blk_body,
                               keep_ref[:, cb:cb + _B])
        keep_ref[:, cb:cb + _B] = kb
        if blk < _KP // _B - 1:
            cnt = _dot(kb, sup_ref[cb:cb + _B, :], ((1,), (0,)))
            keep_ref[...] = keep_ref[...] * (cnt == 0.0).astype(f32)
    keep = keep_ref[...]

    ds_r = jnp.where(rrow < float(_K), srt_t[4:5, :], _NEG)
    ks_r = jnp.where(keep > 0.5, ds_r, _NEG)
    frank = jnp.zeros((1, _KP), f32)
    for rs in range(_NS):
        rcol_s = strip_iota(rs).astype(f32)
        estrip = (strip_iota(rs) == irow).astype(f32)
        keep_c = _dot(estrip, keep, ((1,), (1,)))
        cs = srt[rs * _ST:(rs + 1) * _ST, :]
        ds_c = jnp.where(rcol_s < float(_K), cs[:, 4:5], _NEG)
        ks_c = jnp.where(keep_c > 0.5, ds_c, _NEG)
        bef = ((ks_c > ks_r) | ((ks_c == ks_r) & (rcol_s < rrow))).astype(f32)
        frank = frank + jnp.sum(bef, axis=0, keepdims=True)

    kept_cnt = jnp.sum(keep * (rrow < float(_K)).astype(f32))
    col = jax.lax.broadcasted_iota(jnp.int32, (_ST, 8), 1)
    for rs in range(_OUTP // _ST):
        rcol_s = strip_iota(rs).astype(f32)
        oh = (frank == rcol_s).astype(f32)
        o = _dot(oh, srt, ((1,), (0,)))
        fix = jnp.where(rcol_s < kept_cnt, o, -jnp.inf)
        out_ref[rs * _ST:(rs + 1) * _ST, :] = jnp.where(col == 4, fix, o)


def kernel(boxes, scores):
    sp = jnp.full((_NP,), _NEG, jnp.float32).at[:_N].set(scores)
    s2 = sp.reshape(_ROWS, _LANES)
    bp = jnp.zeros((_NP, 4), jnp.float32).at[:_N].set(boxes)
    idxf = jnp.arange(_NP, dtype=jnp.float32)[:, None]
    payload = jnp.concatenate(
        [bp, sp[:, None], idxf, jnp.zeros((_NP, 2), jnp.float32)], axis=1)
    out = pl.pallas_call(
        _nms_kernel,
        out_shape=jax.ShapeDtypeStruct((_OUTP, 8), jnp.float32),
        scratch_shapes=[pltpu.VMEM((_KP, _KP), jnp.float32),
                        pltpu.VMEM((1, _KP), jnp.float32),
                        pltpu.VMEM((_KP, 8), jnp.float32),
                        pltpu.VMEM((_KP, 8), jnp.float32),
                        pltpu.VMEM((_KP, 128), jnp.float32)],
    )(s2, payload)
    return out[:_OUT, :5]
